# baseline (speedup 1.0000x reference)
import functools

import jax
import jax.numpy as jnp
from jax import lax
from jax.experimental import pallas as pl
from jax.experimental.pallas import tpu as pltpu
from jax.experimental.pallas import tpu_sc as plsc

_EMB = 32
_NCAT = 100
_CATPAD = 128
_IDX_CHUNK = 128


@functools.cache
def _make_sc_gather(vocab_rows: int, emb: int, batch: int):
  info = plsc.get_sparse_core_info()
  nw = info.num_cores * info.num_subcores
  b_per_w = batch // nw
  n_chunks = b_per_w // _IDX_CHUNK
  assert b_per_w * nw == batch and n_chunks * _IDX_CHUNK == b_per_w

  mesh = plsc.VectorSubcoreMesh(core_axis_name="c", subcore_axis_name="s")

  @functools.partial(
      pl.kernel,
      out_type=jax.ShapeDtypeStruct((batch, emb), jnp.float32),
      mesh=mesh,
      scratch_types=[
          pltpu.VMEM((n_chunks, _IDX_CHUNK), jnp.int32),
          pltpu.VMEM((b_per_w, emb), jnp.float32),
          pltpu.SemaphoreType.DMA,
      ],
      compiler_params=pltpu.CompilerParams(use_tc_tiling_on_sc=False),
  )
  def gather(table_hbm, idx_hbm, out_hbm, idx_v, rows_v, sem):
    wid = lax.axis_index("s") * info.num_cores + lax.axis_index("c")
    pltpu.sync_copy(idx_hbm.at[pl.ds(wid * n_chunks, n_chunks)], idx_v)
    copies = []
    for j in range(n_chunks):
      copies.append(
          pltpu.async_copy(
              table_hbm.at[idx_v.at[j]],
              rows_v.at[pl.ds(j * _IDX_CHUNK, _IDX_CHUNK)],
              sem,
          )
      )
    for c in copies:
      c.wait()
    pltpu.sync_copy(rows_v, out_hbm.at[pl.ds(wid * b_per_w, b_per_w)])

  return gather


def _fnn_body(emb_ref, price_ref, cat_ref, w1a_ref, w1p_ref, w1c_ref,
              b1_ref, w2_ref, b2_ref, out_ref):
  bs = emb_ref.shape[0]
  cat = cat_ref[...]
  oh = (cat == lax.broadcasted_iota(jnp.int32, (bs, _CATPAD), 1)).astype(
      jnp.float32)
  x = jnp.dot(emb_ref[...], w1a_ref[...], preferred_element_type=jnp.float32)
  x += jnp.dot(oh, w1c_ref[...], preferred_element_type=jnp.float32)
  x += price_ref[...] * w1p_ref[...] + b1_ref[...]
  h = jnp.maximum(x, 0.0)
  out_ref[...] = (
      jnp.dot(h, w2_ref[...], preferred_element_type=jnp.float32) + b2_ref[...]
  )


def _fnn(item_emb, price, cat_idx, w1a, w1p, w1c, b1, w2, b2, block_rows):
  batch = item_emb.shape[0]
  grid = (batch // block_rows,)
  full = lambda shape: pl.BlockSpec(shape, lambda i: (0, 0))
  return pl.pallas_call(
      _fnn_body,
      grid=grid,
      in_specs=[
          pl.BlockSpec((block_rows, _EMB), lambda i: (i, 0)),
          pl.BlockSpec((block_rows, 1), lambda i: (i, 0)),
          pl.BlockSpec((block_rows, 1), lambda i: (i, 0)),
          full((_EMB, _EMB)),
          full((1, _EMB)),
          full((_CATPAD, _EMB)),
          full((1, _EMB)),
          full((_EMB, _EMB)),
          full((1, _EMB)),
      ],
      out_specs=pl.BlockSpec((block_rows, _EMB), lambda i: (i, 0)),
      out_shape=jax.ShapeDtypeStruct((batch, _EMB), jnp.float32),
  )(item_emb, price, cat_idx, w1a, w1p, w1c, b1, w2, b2)


def kernel(item_idx, category_idx, price, emb_table, W1, b1, W2, b2):
  batch = item_idx.shape[0]
  idx2d = item_idx.reshape(batch // _IDX_CHUNK, _IDX_CHUNK)
  item_emb = _make_sc_gather(emb_table.shape[0], _EMB, batch)(
      emb_table, idx2d)
  w1a = W1[:_EMB]
  w1p = W1[_EMB:_EMB + 1]
  w1c = jnp.pad(W1[_EMB + 1:], ((0, _CATPAD - _NCAT), (0, 0)))
  return _fnn(
      item_emb,
      price[:, None],
      category_idx[:, None],
      w1a, w1p, w1c,
      b1[None],
      W2,
      b2[None],
      block_rows=2048,
  )


# speedup vs baseline: 1.6238x; 1.6238x over previous
import functools

import jax
import jax.numpy as jnp
from jax import lax
from jax.experimental import pallas as pl
from jax.experimental.pallas import tpu as pltpu
from jax.experimental.pallas import tpu_sc as plsc

_EMB = 32
_NCAT = 100
_CATPAD = 128
_IDX_CHUNK = 128


@functools.cache
def _make_sc_gather(vocab_rows: int, emb: int, batch: int):
  info = plsc.get_sparse_core_info()
  nw = info.num_cores * info.num_subcores
  b_per_w = batch // nw
  n_chunks = b_per_w // _IDX_CHUNK
  assert b_per_w * nw == batch and n_chunks * _IDX_CHUNK == b_per_w

  mesh = plsc.VectorSubcoreMesh(core_axis_name="c", subcore_axis_name="s")

  @functools.partial(
      pl.kernel,
      out_type=jax.ShapeDtypeStruct((batch, emb), jnp.float32),
      mesh=mesh,
      scratch_types=[
          pltpu.VMEM((b_per_w,), jnp.int32),
          pltpu.VMEM((b_per_w, emb), jnp.float32),
          pltpu.SemaphoreType.DMA,
      ],
  )
  def gather(table_hbm, idx_hbm, out_hbm, idx_v, rows_v, sem):
    wid = lax.axis_index("s") * info.num_cores + lax.axis_index("c")
    pltpu.sync_copy(idx_hbm.at[pl.ds(wid * b_per_w, b_per_w)], idx_v)

    def issue(g, _):
      v = idx_v[pl.ds(g * 16, 16)]
      for j in range(16):
        pltpu.async_copy(
            table_hbm.at[pl.ds(v[j], 1)],
            rows_v.at[pl.ds(g * 16 + j, 1)],
            sem,
        )
      return 0

    lax.fori_loop(0, b_per_w // 16, issue, 0)
    pltpu.make_async_copy(
        table_hbm.at[pl.ds(0, b_per_w)], rows_v, sem).wait()
    pltpu.sync_copy(rows_v, out_hbm.at[pl.ds(wid * b_per_w, b_per_w)])

  return gather


def _fnn_body(emb_ref, price_ref, cat_ref, w1a_ref, w1p_ref, w1c_ref,
              b1_ref, w2_ref, b2_ref, out_ref):
  bs = emb_ref.shape[0]
  cat = cat_ref[...]
  oh = (cat == lax.broadcasted_iota(jnp.int32, (bs, _CATPAD), 1)).astype(
      jnp.float32)
  x = jnp.dot(emb_ref[...], w1a_ref[...], preferred_element_type=jnp.float32)
  x += jnp.dot(oh, w1c_ref[...], preferred_element_type=jnp.float32)
  x += price_ref[...] * w1p_ref[...] + b1_ref[...]
  h = jnp.maximum(x, 0.0)
  out_ref[...] = (
      jnp.dot(h, w2_ref[...], preferred_element_type=jnp.float32) + b2_ref[...]
  )


def _fnn(item_emb, price, cat_idx, w1a, w1p, w1c, b1, w2, b2, block_rows):
  batch = item_emb.shape[0]
  grid = (batch // block_rows,)
  full = lambda shape: pl.BlockSpec(shape, lambda i: (0, 0))
  return pl.pallas_call(
      _fnn_body,
      grid=grid,
      in_specs=[
          pl.BlockSpec((block_rows, _EMB), lambda i: (i, 0)),
          pl.BlockSpec((block_rows, 1), lambda i: (i, 0)),
          pl.BlockSpec((block_rows, 1), lambda i: (i, 0)),
          full((_EMB, _EMB)),
          full((1, _EMB)),
          full((_CATPAD, _EMB)),
          full((1, _EMB)),
          full((_EMB, _EMB)),
          full((1, _EMB)),
      ],
      out_specs=pl.BlockSpec((block_rows, _EMB), lambda i: (i, 0)),
      out_shape=jax.ShapeDtypeStruct((batch, _EMB), jnp.float32),
  )(item_emb, price, cat_idx, w1a, w1p, w1c, b1, w2, b2)


def kernel(item_idx, category_idx, price, emb_table, W1, b1, W2, b2):
  batch = item_idx.shape[0]
  item_emb = _make_sc_gather(emb_table.shape[0], _EMB, batch)(
      emb_table, item_idx)
  w1a = W1[:_EMB]
  w1p = W1[_EMB:_EMB + 1]
  w1c = jnp.pad(W1[_EMB + 1:], ((0, _CATPAD - _NCAT), (0, 0)))
  return _fnn(
      item_emb,
      price[:, None],
      category_idx[:, None],
      w1a, w1p, w1c,
      b1[None],
      W2,
      b2[None],
      block_rows=2048,
  )


# speedup vs baseline: 2.8834x; 1.7757x over previous
import functools

import jax
import jax.numpy as jnp
from jax import lax
from jax.experimental import pallas as pl
from jax.experimental.pallas import tpu as pltpu
from jax.experimental.pallas import tpu_sc as plsc

_EMB = 32
_NCAT = 100
_CATPAD = 128
_LANES = 16
_GRP = 16


@functools.cache
def _make_sc_gather(vocab_rows: int, emb: int, batch: int):
  info = plsc.get_sparse_core_info()
  nw = info.num_cores * info.num_subcores
  b_per_w = batch // nw
  n_grp = b_per_w // _GRP
  assert b_per_w * nw == batch and n_grp * _GRP == b_per_w

  mesh = plsc.VectorSubcoreMesh(core_axis_name="c", subcore_axis_name="s")

  @functools.partial(
      pl.kernel,
      out_type=jax.ShapeDtypeStruct((batch, emb), jnp.float32),
      mesh=mesh,
      scratch_types=[
          pltpu.VMEM((b_per_w,), jnp.int32),
          pltpu.VMEM((_GRP // 2, emb, 128), jnp.float32),
          pltpu.VMEM((b_per_w, emb), jnp.float32),
          pltpu.SemaphoreType.DMA,
      ],
      compiler_params=pltpu.CompilerParams(needs_layout_passes=False),
  )
  def gather(table_hbm, idx_hbm, out_hbm, idx_v, stk_v, rows_v, sem):
    wid = lax.axis_index("s") * info.num_cores + lax.axis_index("c")
    pltpu.sync_copy(idx_hbm.at[pl.ds(wid * b_per_w, b_per_w)], idx_v)

    iota0 = lax.iota(jnp.int32, _LANES)
    iota1 = iota0 + _LANES

    def group(g, _):
      v = idx_v[pl.ds(g * _GRP, _GRP)]
      col = v & 127
      base = v - col
      for wave in range(2):
        copies = []
        for k in range(_GRP // 2):
          off = pl.multiple_of(base[wave * 8 + k], 128)
          copies.append(
              pltpu.async_copy(
                  table_hbm.at[:, pl.ds(off, 128)], stk_v.at[k], sem))
        for c in copies:
          c.wait()
        for k in range(_GRP // 2):
          ck = jnp.full((_LANES,), col[wave * 8 + k], jnp.int32)
          row = jnp.full((_LANES,), g * _GRP + wave * 8 + k, jnp.int32)
          lo = plsc.load_gather(stk_v.at[k], [iota0, ck])
          hi = plsc.load_gather(stk_v.at[k], [iota1, ck])
          plsc.store_scatter(rows_v, [row, iota0], lo)
          plsc.store_scatter(rows_v, [row, iota1], hi)
      return 0

    lax.fori_loop(0, n_grp, group, 0)
    pltpu.sync_copy(rows_v, out_hbm.at[pl.ds(wid * b_per_w, b_per_w)])

  return gather


def _fnn_body(emb_ref, price_ref, cat_ref, w1a_ref, w1p_ref, w1c_ref,
              b1_ref, w2_ref, b2_ref, out_ref):
  bs = emb_ref.shape[0]
  cat = cat_ref[...]
  oh = (cat == lax.broadcasted_iota(jnp.int32, (bs, _CATPAD), 1)).astype(
      jnp.float32)
  x = jnp.dot(emb_ref[...], w1a_ref[...], preferred_element_type=jnp.float32)
  x += jnp.dot(oh, w1c_ref[...], preferred_element_type=jnp.float32)
  x += price_ref[...] * w1p_ref[...] + b1_ref[...]
  h = jnp.maximum(x, 0.0)
  out_ref[...] = (
      jnp.dot(h, w2_ref[...], preferred_element_type=jnp.float32) + b2_ref[...]
  )


def _fnn(item_emb, price, cat_idx, w1a, w1p, w1c, b1, w2, b2, block_rows):
  batch = item_emb.shape[0]
  grid = (batch // block_rows,)
  full = lambda shape: pl.BlockSpec(shape, lambda i: (0, 0))
  return pl.pallas_call(
      _fnn_body,
      grid=grid,
      in_specs=[
          pl.BlockSpec((block_rows, _EMB), lambda i: (i, 0)),
          pl.BlockSpec((block_rows, 1), lambda i: (i, 0)),
          pl.BlockSpec((block_rows, 1), lambda i: (i, 0)),
          full((_EMB, _EMB)),
          full((1, _EMB)),
          full((_CATPAD, _EMB)),
          full((1, _EMB)),
          full((_EMB, _EMB)),
          full((1, _EMB)),
      ],
      out_specs=pl.BlockSpec((block_rows, _EMB), lambda i: (i, 0)),
      out_shape=jax.ShapeDtypeStruct((batch, _EMB), jnp.float32),
  )(item_emb, price, cat_idx, w1a, w1p, w1c, b1, w2, b2)


def kernel(item_idx, category_idx, price, emb_table, W1, b1, W2, b2):
  batch = item_idx.shape[0]
  item_emb = _make_sc_gather(emb_table.shape[0], _EMB, batch)(
      emb_table.T, item_idx)
  w1a = W1[:_EMB]
  w1p = W1[_EMB:_EMB + 1]
  w1c = jnp.pad(W1[_EMB + 1:], ((0, _CATPAD - _NCAT), (0, 0)))
  return _fnn(
      item_emb,
      price[:, None],
      category_idx[:, None],
      w1a, w1p, w1c,
      b1[None],
      W2,
      b2[None],
      block_rows=2048,
  )


# speedup vs baseline: 3.4294x; 1.1894x over previous
import functools

import jax
import jax.numpy as jnp
from jax import lax
from jax.experimental import pallas as pl
from jax.experimental.pallas import tpu as pltpu
from jax.experimental.pallas import tpu_sc as plsc

_EMB = 32
_NCAT = 100
_CATPAD = 128
_LANES = 16
_GRP = 16


@functools.cache
def _make_sc_gather(vocab_rows: int, emb: int, batch: int):
  info = plsc.get_sparse_core_info()
  nw = info.num_cores * info.num_subcores
  b_per_w = batch // nw
  n_grp = b_per_w // _GRP
  assert b_per_w * nw == batch and n_grp * _GRP == b_per_w

  mesh = plsc.VectorSubcoreMesh(core_axis_name="c", subcore_axis_name="s")

  wave_sz = 8
  n_wave = b_per_w // wave_sz
  rows_cap = 128
  waves_per_flush = rows_cap // wave_sz

  @functools.partial(
      pl.kernel,
      out_type=jax.ShapeDtypeStruct((batch, emb), jnp.float32),
      mesh=mesh,
      scratch_types=[
          pltpu.VMEM((b_per_w + _LANES,), jnp.int32),
          pltpu.VMEM((2, wave_sz, emb, 128), jnp.float32),
          pltpu.VMEM((rows_cap, emb), jnp.float32),
          pltpu.SemaphoreType.DMA((2,)),
      ],
      compiler_params=pltpu.CompilerParams(needs_layout_passes=False),
  )
  def gather(table_hbm, idx_hbm, out_hbm, idx_v, stk_v, rows_v, sems):
    wid = lax.axis_index("s") * info.num_cores + lax.axis_index("c")
    pltpu.sync_copy(
        idx_hbm.at[pl.ds(wid * b_per_w, b_per_w)],
        idx_v.at[pl.ds(0, b_per_w)])

    iota0 = lax.iota(jnp.int32, _LANES)
    iota1 = iota0 + _LANES

    def issue(w, p):
      v = idx_v[pl.ds(w * wave_sz, _LANES)]
      base = v - (v & 127)
      for k in range(wave_sz):
        off = pl.multiple_of(base[k], 128)
        pltpu.async_copy(
            table_hbm.at[:, pl.ds(off, 128)], stk_v.at[p, k], sems.at[p])

    def drain(p):
      for k in range(wave_sz):
        pltpu.make_async_copy(
            table_hbm.at[:, pl.ds(0, 128)], stk_v.at[p, k], sems.at[p]).wait()

    def extract(w, p):
      v = idx_v[pl.ds(w * wave_sz, _LANES)]
      col = v & 127
      for k in range(wave_sz):
        ck = jnp.full((_LANES,), col[k], jnp.int32)
        row = jnp.full((_LANES,), (w % waves_per_flush) * wave_sz + k,
                       jnp.int32)
        lo = plsc.load_gather(stk_v.at[p, k], [iota0, ck])
        hi = plsc.load_gather(stk_v.at[p, k], [iota1, ck])
        plsc.store_scatter(rows_v, [row, iota0], lo)
        plsc.store_scatter(rows_v, [row, iota1], hi)

    issue(0, 0)

    def body(w, _):
      p = lax.rem(w, 2)

      @pl.when(w + 1 < n_wave)
      def _():
        issue(w + 1, 1 - p)

      drain(p)
      extract(w, p)

      @pl.when(lax.rem(w, waves_per_flush) == waves_per_flush - 1)
      def _():
        pltpu.sync_copy(
            rows_v,
            out_hbm.at[pl.ds(
                wid * b_per_w + (w + 1 - waves_per_flush) * wave_sz,
                rows_cap)])

      return 0

    lax.fori_loop(0, n_wave, body, 0)

  return gather


def _fnn_body(emb_ref, price_ref, cat_ref, w1a_ref, w1p_ref, w1c_ref,
              b1_ref, w2_ref, b2_ref, out_ref):
  bs = emb_ref.shape[0]
  cat = cat_ref[...]
  oh = (cat == lax.broadcasted_iota(jnp.int32, (bs, _CATPAD), 1)).astype(
      jnp.float32)
  x = jnp.dot(emb_ref[...], w1a_ref[...], preferred_element_type=jnp.float32)
  x += jnp.dot(oh, w1c_ref[...], preferred_element_type=jnp.float32)
  x += price_ref[...] * w1p_ref[...] + b1_ref[...]
  h = jnp.maximum(x, 0.0)
  out_ref[...] = (
      jnp.dot(h, w2_ref[...], preferred_element_type=jnp.float32) + b2_ref[...]
  )


def _fnn(item_emb, price, cat_idx, w1a, w1p, w1c, b1, w2, b2, block_rows):
  batch = item_emb.shape[0]
  grid = (batch // block_rows,)
  full = lambda shape: pl.BlockSpec(shape, lambda i: (0, 0))
  return pl.pallas_call(
      _fnn_body,
      grid=grid,
      in_specs=[
          pl.BlockSpec((block_rows, _EMB), lambda i: (i, 0)),
          pl.BlockSpec((block_rows, 1), lambda i: (i, 0)),
          pl.BlockSpec((block_rows, 1), lambda i: (i, 0)),
          full((_EMB, _EMB)),
          full((1, _EMB)),
          full((_CATPAD, _EMB)),
          full((1, _EMB)),
          full((_EMB, _EMB)),
          full((1, _EMB)),
      ],
      out_specs=pl.BlockSpec((block_rows, _EMB), lambda i: (i, 0)),
      out_shape=jax.ShapeDtypeStruct((batch, _EMB), jnp.float32),
  )(item_emb, price, cat_idx, w1a, w1p, w1c, b1, w2, b2)


def kernel(item_idx, category_idx, price, emb_table, W1, b1, W2, b2):
  batch = item_idx.shape[0]
  item_emb = _make_sc_gather(emb_table.shape[0], _EMB, batch)(
      emb_table.T, item_idx)
  w1a = W1[:_EMB]
  w1p = W1[_EMB:_EMB + 1]
  w1c = jnp.pad(W1[_EMB + 1:], ((0, _CATPAD - _NCAT), (0, 0)))
  return _fnn(
      item_emb,
      price[:, None],
      category_idx[:, None],
      w1a, w1p, w1c,
      b1[None],
      W2,
      b2[None],
      block_rows=2048,
  )


# speedup vs baseline: 3.8332x; 1.1177x over previous
import functools

import jax
import jax.numpy as jnp
from jax import lax
from jax.experimental import pallas as pl
from jax.experimental.pallas import tpu as pltpu
from jax.experimental.pallas import tpu_sc as plsc

_EMB = 32
_NCAT = 100
_CATPAD = 128
_LANES = 16
_GRP = 16


@functools.cache
def _make_sc_gather(vocab_rows: int, emb: int, batch: int):
  info = plsc.get_sparse_core_info()
  nw = info.num_cores * info.num_subcores
  b_per_w = batch // nw
  n_grp = b_per_w // _GRP
  assert b_per_w * nw == batch and n_grp * _GRP == b_per_w

  mesh = plsc.VectorSubcoreMesh(core_axis_name="c", subcore_axis_name="s")

  wave_sz = 8
  n_wave = b_per_w // wave_sz
  rows_cap = 128
  waves_per_flush = rows_cap // wave_sz

  @functools.partial(
      pl.kernel,
      out_type=jax.ShapeDtypeStruct((emb, batch), jnp.float32),
      mesh=mesh,
      scratch_types=[
          pltpu.VMEM((b_per_w + _LANES,), jnp.int32),
          pltpu.VMEM((2, wave_sz, emb, 128), jnp.float32),
          pltpu.VMEM((emb, rows_cap), jnp.float32),
          pltpu.SemaphoreType.DMA((2,)),
      ],
      compiler_params=pltpu.CompilerParams(needs_layout_passes=False),
  )
  def gather(table_hbm, idx_hbm, out_hbm, idx_v, stk_v, cols_v, sems):
    wid = lax.axis_index("s") * info.num_cores + lax.axis_index("c")
    pltpu.sync_copy(
        idx_hbm.at[pl.ds(wid * b_per_w, b_per_w)],
        idx_v.at[pl.ds(0, b_per_w)])

    iota0 = lax.iota(jnp.int32, _LANES)
    iota1 = iota0 + _LANES

    def issue(w, p):
      v = idx_v[pl.ds(w * wave_sz, _LANES)]
      base = v - (v & 127)
      for k in range(wave_sz):
        off = pl.multiple_of(base[k], 128)
        pltpu.async_copy(
            table_hbm.at[:, pl.ds(off, 128)], stk_v.at[p, k], sems.at[p])

    def drain(p):
      for k in range(wave_sz):
        pltpu.make_async_copy(
            table_hbm.at[:, pl.ds(0, 128)], stk_v.at[p, k], sems.at[p]).wait()

    def extract(w, p):
      v = idx_v[pl.ds(w * wave_sz, _LANES)]
      col = v & 127
      for k in range(wave_sz):
        ck = jnp.full((_LANES,), col[k], jnp.int32)
        dst = jnp.full((_LANES,), (w % waves_per_flush) * wave_sz + k,
                       jnp.int32)
        lo = plsc.load_gather(stk_v.at[p, k], [iota0, ck])
        hi = plsc.load_gather(stk_v.at[p, k], [iota1, ck])
        plsc.store_scatter(cols_v, [iota0, dst], lo)
        plsc.store_scatter(cols_v, [iota1, dst], hi)

    issue(0, 0)

    def body(w, _):
      p = lax.rem(w, 2)

      @pl.when(w + 1 < n_wave)
      def _():
        issue(w + 1, 1 - p)

      drain(p)
      extract(w, p)

      @pl.when(lax.rem(w, waves_per_flush) == waves_per_flush - 1)
      def _():
        col_base = pl.multiple_of(
            wid * b_per_w + (w + 1 - waves_per_flush) * wave_sz, 128)
        pltpu.sync_copy(cols_v, out_hbm.at[:, pl.ds(col_base, rows_cap)])

      return 0

    lax.fori_loop(0, n_wave, body, 0)

  return gather


def _fnn_body(emb_ref, price_ref, cat_ref, w1a_ref, w1p_ref, w1c_ref,
              b1_ref, w2_ref, b2_ref, out_ref):
  bs = emb_ref.shape[1]
  cat = cat_ref[...]
  oh_t = (cat == lax.broadcasted_iota(jnp.int32, (_CATPAD, bs), 0)).astype(
      jnp.float32)
  dn = (((0,), (0,)), ((), ()))
  x = lax.dot_general(w1a_ref[...], emb_ref[...], dn,
                      preferred_element_type=jnp.float32)
  x += lax.dot_general(w1c_ref[...], oh_t, dn,
                       preferred_element_type=jnp.float32)
  x += w1p_ref[...] * price_ref[...] + b1_ref[...]
  h = jnp.maximum(x, 0.0)
  out_ref[...] = (
      lax.dot_general(w2_ref[...], h, dn,
                      preferred_element_type=jnp.float32) + b2_ref[...]
  )


def _fnn(emb_t, price_row, cat_row, w1a, w1p_col, w1c, b1_col, w2, b2_col,
         block_cols):
  batch = emb_t.shape[1]
  grid = (batch // block_cols,)
  full = lambda shape: pl.BlockSpec(shape, lambda i: (0, 0))
  return pl.pallas_call(
      _fnn_body,
      grid=grid,
      in_specs=[
          pl.BlockSpec((_EMB, block_cols), lambda i: (0, i)),
          pl.BlockSpec((1, block_cols), lambda i: (0, i)),
          pl.BlockSpec((1, block_cols), lambda i: (0, i)),
          full((_EMB, _EMB)),
          full((_EMB, 1)),
          full((_CATPAD, _EMB)),
          full((_EMB, 1)),
          full((_EMB, _EMB)),
          full((_EMB, 1)),
      ],
      out_specs=pl.BlockSpec((_EMB, block_cols), lambda i: (0, i)),
      out_shape=jax.ShapeDtypeStruct((_EMB, batch), jnp.float32),
  )(emb_t, price_row, cat_row, w1a, w1p_col, w1c, b1_col, w2, b2_col)


def kernel(item_idx, category_idx, price, emb_table, W1, b1, W2, b2):
  batch = item_idx.shape[0]
  emb_t = _make_sc_gather(emb_table.shape[0], _EMB, batch)(
      emb_table.T, item_idx)
  w1a = W1[:_EMB]
  w1p_col = W1[_EMB:_EMB + 1].T
  w1c = jnp.pad(W1[_EMB + 1:], ((0, _CATPAD - _NCAT), (0, 0)))
  out_t = _fnn(
      emb_t,
      price[None, :],
      category_idx[None, :],
      w1a, w1p_col, w1c,
      b1[:, None],
      W2,
      b2[:, None],
      block_cols=2048,
  )
  return out_t.T


# speedup vs baseline: 4.0533x; 1.0574x over previous
import functools

import jax
import jax.numpy as jnp
from jax import lax
from jax.experimental import pallas as pl
from jax.experimental.pallas import tpu as pltpu
from jax.experimental.pallas import tpu_sc as plsc

_EMB = 32
_NCAT = 100
_CATPAD = 128
_LANES = 16
_GRP = 16


@functools.cache
def _make_sc_gather(vocab_rows: int, emb: int, batch: int):
  info = plsc.get_sparse_core_info()
  nw = info.num_cores * info.num_subcores
  b_per_w = batch // nw
  n_grp = b_per_w // _GRP
  assert b_per_w * nw == batch and n_grp * _GRP == b_per_w

  mesh = plsc.VectorSubcoreMesh(core_axis_name="c", subcore_axis_name="s")

  n_slot = _GRP
  rows_cap = 128
  grp_per_flush = rows_cap // _GRP

  @functools.partial(
      pl.kernel,
      out_type=jax.ShapeDtypeStruct((emb, batch), jnp.float32),
      mesh=mesh,
      scratch_types=[
          pltpu.VMEM((b_per_w + _GRP,), jnp.int32),
          pltpu.VMEM((n_slot, emb, 128), jnp.float32),
          pltpu.VMEM((emb, rows_cap), jnp.float32),
          pltpu.SemaphoreType.DMA((n_slot,)),
      ],
      compiler_params=pltpu.CompilerParams(needs_layout_passes=False),
  )
  def gather(table_hbm, idx_hbm, out_hbm, idx_v, stk_v, cols_v, sems):
    wid = lax.axis_index("s") * info.num_cores + lax.axis_index("c")
    pltpu.sync_copy(
        idx_hbm.at[pl.ds(wid * b_per_w, b_per_w)],
        idx_v.at[pl.ds(0, b_per_w)])

    iota0 = lax.iota(jnp.int32, _LANES)
    iota1 = iota0 + _LANES

    def issue(k, base):
      off = pl.multiple_of(base[k], 128)
      pltpu.async_copy(
          table_hbm.at[:, pl.ds(off, 128)], stk_v.at[k], sems.at[k])

    v0 = idx_v[pl.ds(0, _GRP)]
    base0 = v0 - (v0 & 127)
    for k in range(n_slot):
      issue(k, base0)

    def group(g, _):
      v = idx_v[pl.ds(g * _GRP, _GRP)]
      col = v & 127
      vn = idx_v[pl.ds((g + 1) * _GRP, _GRP)]
      base_n = vn - (vn & 127)
      for k in range(n_slot):
        pltpu.make_async_copy(
            table_hbm.at[:, pl.ds(0, 128)], stk_v.at[k], sems.at[k]).wait()
        ck = jnp.full((_LANES,), col[k], jnp.int32)
        dst = jnp.full((_LANES,), lax.rem(g, grp_per_flush) * _GRP + k,
                       jnp.int32)
        lo = plsc.load_gather(stk_v.at[k], [iota0, ck])
        hi = plsc.load_gather(stk_v.at[k], [iota1, ck])
        plsc.store_scatter(cols_v, [iota0, dst], lo)
        plsc.store_scatter(cols_v, [iota1, dst], hi)

        @pl.when(g + 1 < n_grp)
        def _():
          issue(k, base_n)

      @pl.when(lax.rem(g, grp_per_flush) == grp_per_flush - 1)
      def _():
        col_base = pl.multiple_of(
            wid * b_per_w + (g + 1 - grp_per_flush) * _GRP, 128)
        pltpu.sync_copy(cols_v, out_hbm.at[:, pl.ds(col_base, rows_cap)])

      return 0

    lax.fori_loop(0, n_grp, group, 0)

  return gather


def _fnn_body(emb_ref, price_ref, cat_ref, w1a_ref, w1p_ref, w1c_ref,
              b1_ref, w2_ref, b2_ref, out_ref):
  bs = emb_ref.shape[1]
  cat = cat_ref[...]
  oh_t = (cat == lax.broadcasted_iota(jnp.int32, (_CATPAD, bs), 0)).astype(
      jnp.float32)
  dn = (((0,), (0,)), ((), ()))
  x = lax.dot_general(w1a_ref[...], emb_ref[...], dn,
                      preferred_element_type=jnp.float32)
  x += lax.dot_general(w1c_ref[...], oh_t, dn,
                       preferred_element_type=jnp.float32)
  x += w1p_ref[...] * price_ref[...] + b1_ref[...]
  h = jnp.maximum(x, 0.0)
  out_ref[...] = (
      lax.dot_general(w2_ref[...], h, dn,
                      preferred_element_type=jnp.float32) + b2_ref[...]
  )


def _fnn(emb_t, price_row, cat_row, w1a, w1p_col, w1c, b1_col, w2, b2_col,
         block_cols):
  batch = emb_t.shape[1]
  grid = (batch // block_cols,)
  full = lambda shape: pl.BlockSpec(shape, lambda i: (0, 0))
  return pl.pallas_call(
      _fnn_body,
      grid=grid,
      in_specs=[
          pl.BlockSpec((_EMB, block_cols), lambda i: (0, i)),
          pl.BlockSpec((1, block_cols), lambda i: (0, i)),
          pl.BlockSpec((1, block_cols), lambda i: (0, i)),
          full((_EMB, _EMB)),
          full((_EMB, 1)),
          full((_CATPAD, _EMB)),
          full((_EMB, 1)),
          full((_EMB, _EMB)),
          full((_EMB, 1)),
      ],
      out_specs=pl.BlockSpec((_EMB, block_cols), lambda i: (0, i)),
      out_shape=jax.ShapeDtypeStruct((_EMB, batch), jnp.float32),
  )(emb_t, price_row, cat_row, w1a, w1p_col, w1c, b1_col, w2, b2_col)


def kernel(item_idx, category_idx, price, emb_table, W1, b1, W2, b2):
  batch = item_idx.shape[0]
  emb_t = _make_sc_gather(emb_table.shape[0], _EMB, batch)(
      emb_table.T, item_idx)
  w1a = W1[:_EMB]
  w1p_col = W1[_EMB:_EMB + 1].T
  w1c = jnp.pad(W1[_EMB + 1:], ((0, _CATPAD - _NCAT), (0, 0)))
  out_t = _fnn(
      emb_t,
      price[None, :],
      category_idx[None, :],
      w1a, w1p_col, w1c,
      b1[:, None],
      W2,
      b2[:, None],
      block_cols=2048,
  )
  return out_t.T
